# Initial kernel scaffold; baseline (speedup 1.0000x reference)
import jax
import jax.numpy as jnp
from jax.experimental import pallas as pl


def kernel(token_ids, embedding_table):
    raise NotImplementedError("write your pallas kernel here")


# speedup vs baseline: 1.0222x; 1.0222x over previous
import functools

import jax
import jax.numpy as jnp
from jax import lax
from jax.experimental import pallas as pl
from jax.experimental.pallas import tpu as pltpu
from jax.experimental.pallas import tpu_sc as plsc

NUM_EMB = 1000000
DIM = 32
BATCH = 16384
SEQ = 50
TOTAL = BATCH * SEQ

NC = 2
NS = 16
NW = NC * NS
PER_W = TOTAL // NW
G = 128
STEPS = PER_W // G

_mesh = plsc.VectorSubcoreMesh(core_axis_name="c", subcore_axis_name="s")


@functools.partial(
    pl.kernel,
    out_type=jax.ShapeDtypeStruct((TOTAL, DIM), jnp.float32),
    mesh=_mesh,
    scratch_types=[
        pltpu.VMEM((STEPS, G), jnp.int32),
        pltpu.VMEM((G, DIM), jnp.float32),
        pltpu.SemaphoreType.DMA,
    ],
    compiler_params=pltpu.CompilerParams(use_tc_tiling_on_sc=False),
)
def _emb_lookup(idx_hbm, table_hbm, out_hbm, idx_v, rows_v, sem):
    wid = lax.axis_index("s") * NC + lax.axis_index("c")
    base = wid * PER_W
    pltpu.sync_copy(idx_hbm.at[wid], idx_v)

    def step(j, carry):
        pltpu.async_copy(table_hbm.at[idx_v.at[j]], rows_v, sem).wait()
        pltpu.sync_copy(rows_v, out_hbm.at[pl.ds(base + j * G, G)])
        return carry

    lax.fori_loop(0, STEPS, step, 0)


def kernel(token_ids, embedding_table):
    idx = token_ids.reshape(NW, STEPS, G).astype(jnp.int32)
    out = _emb_lookup(idx, embedding_table)
    return out.reshape(BATCH, SEQ, DIM)


# speedup vs baseline: 1.1127x; 1.0885x over previous
import functools

import jax
import jax.numpy as jnp
from jax import lax
from jax.experimental import pallas as pl
from jax.experimental.pallas import tpu as pltpu
from jax.experimental.pallas import tpu_sc as plsc

NUM_EMB = 1000000
DIM = 32
BATCH = 16384
SEQ = 50
TOTAL = BATCH * SEQ

NC = 2
NS = 16
NW = NC * NS
PER_W = TOTAL // NW
G = 128
STEPS = PER_W // G
NBUF = 8

_mesh = plsc.VectorSubcoreMesh(core_axis_name="c", subcore_axis_name="s")


@functools.partial(
    pl.kernel,
    out_type=jax.ShapeDtypeStruct((TOTAL, DIM), jnp.float32),
    mesh=_mesh,
    scratch_types=[
        pltpu.VMEM((STEPS, G), jnp.int32),
        pltpu.VMEM((NBUF, G, DIM), jnp.float32),
        pltpu.SemaphoreType.DMA,
    ],
    compiler_params=pltpu.CompilerParams(use_tc_tiling_on_sc=False),
)
def _emb_lookup(idx_hbm, table_hbm, out_hbm, idx_v, rows_v, gsem):
    wid = lax.axis_index("s") * NC + lax.axis_index("c")
    base = wid * PER_W
    pltpu.sync_copy(idx_hbm.at[wid], idx_v)

    for b in range(NBUF):
        pltpu.async_copy(table_hbm.at[idx_v.at[b]], rows_v.at[b], gsem)

    @pl.loop(0, STEPS - NBUF, step=NBUF)
    def _main(jj):
        for b in range(NBUF):
            pltpu.make_async_copy(
                table_hbm.at[idx_v.at[b]], rows_v.at[b], gsem
            ).wait()
            pltpu.sync_copy(rows_v.at[b], out_hbm.at[pl.ds(base + (jj + b) * G, G)])
            pltpu.async_copy(
                table_hbm.at[idx_v.at[jj + NBUF + b]], rows_v.at[b], gsem
            )

    for b in range(NBUF):
        j = STEPS - NBUF + b
        pltpu.make_async_copy(
            table_hbm.at[idx_v.at[b]], rows_v.at[b], gsem
        ).wait()
        pltpu.sync_copy(rows_v.at[b], out_hbm.at[pl.ds(base + j * G, G)])


def kernel(token_ids, embedding_table):
    idx = token_ids.reshape(NW, STEPS, G).astype(jnp.int32)
    out = _emb_lookup(idx, embedding_table)
    return out.reshape(BATCH, SEQ, DIM)


# speedup vs baseline: 1.5443x; 1.3879x over previous
import functools

import jax
import jax.numpy as jnp
from jax import lax
from jax.experimental import pallas as pl
from jax.experimental.pallas import tpu as pltpu
from jax.experimental.pallas import tpu_sc as plsc

NUM_EMB = 1000000
DIM = 32
BATCH = 16384
SEQ = 50
TOTAL = BATCH * SEQ

NC = 2
NS = 16
NW = NC * NS
PER_W = TOTAL // NW
G = 128
STEPS = PER_W // G
GROUPS_PER_S = BATCH // G
OUT_ROWS = TOTAL * DIM // G

_mesh = plsc.VectorSubcoreMesh(core_axis_name="c", subcore_axis_name="s")


def _transpose_block(rows_v, tr_v):
    lanes = lax.iota(jnp.int32, 16)
    for f in range(DIM):
        col = jnp.full((16,), f, jnp.int32)
        for j in range(G // 16):
            vec = plsc.load_gather(rows_v, [lanes + (16 * j), col])
            tr_v[f, pl.ds(16 * j, 16)] = vec


@functools.partial(
    pl.kernel,
    out_type=jax.ShapeDtypeStruct((OUT_ROWS, G), jnp.float32),
    mesh=_mesh,
    scratch_types=[
        pltpu.VMEM((STEPS, G), jnp.int32),
        pltpu.VMEM((2, G, DIM), jnp.float32),
        pltpu.VMEM((2, DIM, G), jnp.float32),
        pltpu.SemaphoreType.DMA,
        pltpu.SemaphoreType.DMA,
    ],
    compiler_params=pltpu.CompilerParams(
        use_tc_tiling_on_sc=False, needs_layout_passes=False
    ),
)
def _emb_lookup(idx_hbm, table_hbm, out_hbm, idx_v, rows_v, tr_v, gsem, ssem):
    wid = lax.axis_index("s") * NC + lax.axis_index("c")
    g0 = wid * STEPS
    pltpu.sync_copy(idx_hbm.at[wid], idx_v)

    def start_gather(i, slot):
        pltpu.async_copy(table_hbm.at[idx_v.at[i]], rows_v.at[slot], gsem)

    def wait_gather(slot):
        pltpu.make_async_copy(
            table_hbm.at[idx_v.at[0]], rows_v.at[slot], gsem
        ).wait()

    def start_stores(i, slot):
        g = g0 + i
        s = g // GROUPS_PER_S
        tb = g % GROUPS_PER_S
        base = s * 4096 + tb * 8
        for tf in range(4):
            pltpu.async_copy(
                tr_v.at[slot].at[pl.ds(8 * tf, 8)],
                out_hbm.at[pl.ds(base + tf * 1024, 8)],
                ssem,
            )

    def wait_stores(slot):
        for tf in range(4):
            pltpu.make_async_copy(
                tr_v.at[slot].at[pl.ds(8 * tf, 8)],
                out_hbm.at[pl.ds(0, 8)],
                ssem,
            ).wait()

    start_gather(0, 0)

    @pl.loop(0, STEPS, step=2)
    def _main(i):
        for slot in range(2):
            g = i + slot
            wait_gather(slot)

            @pl.when(g + 1 < STEPS)
            def _():
                start_gather(g + 1, 1 - slot)

            @pl.when(g >= 2)
            def _():
                wait_stores(slot)

            _transpose_block(rows_v.at[slot], tr_v.at[slot])
            start_stores(g, slot)

    wait_stores(0)
    wait_stores(1)


def kernel(token_ids, embedding_table):
    idx = jnp.transpose(token_ids).reshape(NW, STEPS, G).astype(jnp.int32)
    out_lin = _emb_lookup(idx, embedding_table)
    x = out_lin.reshape(SEQ, 4, GROUPS_PER_S, 8, G)
    x = jnp.transpose(x, (2, 4, 0, 1, 3))
    return x.reshape(BATCH, SEQ, DIM)


# speedup vs baseline: 1.8644x; 1.2073x over previous
import functools

import jax
import jax.numpy as jnp
from jax import lax
from jax.experimental import pallas as pl
from jax.experimental.pallas import tpu as pltpu
from jax.experimental.pallas import tpu_sc as plsc

NUM_EMB = 1000000
DIM = 32
BATCH = 16384
SEQ = 50
TOTAL = BATCH * SEQ

NC = 2
NS = 16
NW = NC * NS
PER_W = TOTAL // NW
G = 128
STEPS = PER_W // G
GROUPS_PER_S = BATCH // G
OUT_ROWS = TOTAL * DIM // G

_mesh = plsc.VectorSubcoreMesh(core_axis_name="c", subcore_axis_name="s")


def _transpose_block(rows_v, tr_v):
    lanes = lax.iota(jnp.int32, 16)
    c_lo = lanes * G
    c_hi = (lanes + 16) * G

    @pl.loop(0, G, step=8)
    def _t(k0):
        for u in range(8):
            k = k0 + u
            kv = jnp.broadcast_to(k, (16,)).astype(jnp.int32)
            v_lo = rows_v[k, pl.ds(0, 16)]
            v_hi = rows_v[k, pl.ds(16, 16)]
            plsc.store_scatter(tr_v, [c_lo + kv], v_lo)
            plsc.store_scatter(tr_v, [c_hi + kv], v_hi)


@functools.partial(
    pl.kernel,
    out_type=jax.ShapeDtypeStruct((TOTAL * DIM,), jnp.float32),
    mesh=_mesh,
    scratch_types=[
        pltpu.VMEM((STEPS, G), jnp.int32),
        pltpu.VMEM((2, G, DIM), jnp.float32),
        pltpu.VMEM((2, DIM * G), jnp.float32),
        pltpu.SemaphoreType.DMA,
        pltpu.SemaphoreType.DMA,
    ],
    compiler_params=pltpu.CompilerParams(
        use_tc_tiling_on_sc=False, needs_layout_passes=False
    ),
)
def _emb_lookup(idx_hbm, table_hbm, out_hbm, idx_v, rows_v, tr_v, gsem, ssem):
    wid = lax.axis_index("s") * NC + lax.axis_index("c")
    g0 = wid * STEPS
    pltpu.sync_copy(idx_hbm.at[wid], idx_v)

    def start_gather(i, slot):
        pltpu.async_copy(table_hbm.at[idx_v.at[i]], rows_v.at[slot], gsem)

    def wait_gather(slot):
        pltpu.make_async_copy(
            table_hbm.at[idx_v.at[0]], rows_v.at[slot], gsem
        ).wait()

    def start_stores(i, slot):
        g = g0 + i
        s = g // GROUPS_PER_S
        tb = g % GROUPS_PER_S
        base = s * 524288 + tb * 1024
        for tf in range(4):
            pltpu.async_copy(
                tr_v.at[slot].at[pl.ds(tf * 1024, 1024)],
                out_hbm.at[pl.ds(base + tf * 131072, 1024)],
                ssem,
            )

    def wait_stores(slot):
        for tf in range(4):
            pltpu.make_async_copy(
                tr_v.at[slot].at[pl.ds(tf * 1024, 1024)],
                out_hbm.at[pl.ds(0, 1024)],
                ssem,
            ).wait()

    start_gather(0, 0)

    @pl.loop(0, STEPS, step=2)
    def _main(i):
        for slot in range(2):
            g = i + slot
            wait_gather(slot)

            @pl.when(g + 1 < STEPS)
            def _():
                start_gather(g + 1, 1 - slot)

            @pl.when(g >= 2)
            def _():
                wait_stores(slot)

            _transpose_block(rows_v.at[slot], tr_v.at[slot])
            start_stores(g, slot)

    wait_stores(0)
    wait_stores(1)


def kernel(token_ids, embedding_table):
    idx = jnp.transpose(token_ids).reshape(NW, STEPS, G).astype(jnp.int32)
    out_lin = _emb_lookup(idx, embedding_table)
    x = out_lin.reshape(SEQ, 4, GROUPS_PER_S, 8, G)
    x = jnp.transpose(x, (2, 4, 0, 1, 3))
    return x.reshape(BATCH, SEQ, DIM)
